# baseline (speedup 1.0000x reference)
import jax
import jax.numpy as jnp
from jax import lax
from jax.experimental import pallas as pl
from jax.experimental.pallas import tpu as pltpu
from jax.experimental.pallas import tpu_sc as plsc

VOCAB = 1000000
EMBED_DIM = 64
HALF_DIM = EMBED_DIM // 2
BATCH = 16384
HIST = 50

NC = 2
NS = 16
NW = NC * NS
EPW = BATCH // NW
GRP = 2
NGRP = EPW // GRP
NBUF = 4
NLANE = 16
KREG = HALF_DIM // NLANE

VCHUNK = 32768
QQ = VCHUNK // 4
NTBLK = (VOCAB + VCHUNK - 1) // VCHUNK
VOCAB_PAD = NTBLK * VCHUNK


def _tbody(tt_ref, out_ref):
    qs = [tt_ref[:, k * QQ : (k + 1) * QQ].T for k in range(4)]
    out_ref[...] = jnp.concatenate(qs, axis=1)


def _relayout_half(table_t, h):
    return pl.pallas_call(
        _tbody,
        grid=(NTBLK,),
        in_specs=[pl.BlockSpec((HALF_DIM, VCHUNK), lambda i, h=h: (h, i))],
        out_specs=pl.BlockSpec((QQ, 4 * HALF_DIM), lambda i: (i, 0)),
        out_shape=jax.ShapeDtypeStruct((VOCAB_PAD // 4, 4 * HALF_DIM), jnp.float32),
    )(table_t)


def _remap_indices(v):
    i = v // VCHUNK
    r = v & (VCHUNK - 1)
    k = r // QQ
    p = r & (QQ - 1)
    return i * VCHUNK + 4 * p + k


def _body(idx_hbm, table_hbm, out_hbm, idx_v, rows_v, out_v, sems):
    c = lax.axis_index("c")
    s = lax.axis_index("s")
    wid = s * NC + c

    pltpu.sync_copy(idx_hbm.at[wid], idx_v)

    inv = jnp.float32(1.0 / HIST)

    def gather(g, b):
        return pltpu.make_async_copy(
            table_hbm.at[idx_v.at[g]], rows_v.at[b], sems.at[b]
        )

    for b in range(NBUF):
        gather(b, b).start()

    def outer(it, carry):
        for b in range(NBUF):
            g = it * NBUF + b
            gather(g, b).wait()
            for e in range(GRP):
                base = e * HIST
                accs = [
                    rows_v[b, base, pl.ds(k * NLANE, NLANE)] for k in range(KREG)
                ]
                for j in range(1, HIST):
                    for k in range(KREG):
                        accs[k] = accs[k] + rows_v[b, base + j, pl.ds(k * NLANE, NLANE)]
                for k in range(KREG):
                    out_v[g * GRP + e, pl.ds(k * NLANE, NLANE)] = accs[k] * inv

            @pl.when(g + NBUF < NGRP)
            def _():
                gather(g + NBUF, b).start()
        return carry

    lax.fori_loop(0, NGRP // NBUF, outer, 0)

    pltpu.sync_copy(out_v, out_hbm.at[wid])


def _gather_mean(idx3, table_half):
    mesh = plsc.VectorSubcoreMesh(core_axis_name="c", subcore_axis_name="s")
    f = pl.kernel(
        _body,
        out_type=jax.ShapeDtypeStruct((NW, EPW, HALF_DIM), jnp.float32),
        mesh=mesh,
        scratch_types=[
            pltpu.VMEM((NGRP, GRP * HIST), jnp.int32),
            pltpu.VMEM((NBUF, GRP * HIST, HALF_DIM), jnp.float32),
            pltpu.VMEM((EPW, HALF_DIM), jnp.float32),
            pltpu.SemaphoreType.DMA((NBUF,)),
        ],
        compiler_params=pltpu.CompilerParams(use_tc_tiling_on_sc=False),
    )
    return f(idx3, table_half)


@jax.jit
def _run(indices, table):
    idx3 = _remap_indices(indices.astype(jnp.int32)).reshape(NW, NGRP, GRP * HIST)
    tt = table.T
    t0 = _relayout_half(tt, 0).reshape(VOCAB_PAD, HALF_DIM)
    o0 = _gather_mean(idx3, t0)
    t1 = _relayout_half(tt, 1).reshape(VOCAB_PAD, HALF_DIM)
    o1 = _gather_mean(idx3, t1)
    out = jnp.concatenate([o0, o1], axis=-1)
    return out.reshape(BATCH, EMBED_DIM)


def kernel(indices, table):
    return _run(indices, table)


# speedup vs baseline: 1.6444x; 1.6444x over previous
import jax
import jax.numpy as jnp
from jax import lax
from jax.experimental import pallas as pl
from jax.experimental.pallas import tpu as pltpu
from jax.experimental.pallas import tpu_sc as plsc

VOCAB = 1000000
EMBED_DIM = 64
HALF_DIM = EMBED_DIM // 2
BATCH = 16384
HIST = 50

NC = 2
NS = 16
NW = NC * NS
EPW = BATCH // NW
GRP = 2
NGRP = EPW // GRP
NBUF = 4
NLANE = 16
KREG = HALF_DIM // NLANE

VCHUNK = 8192
PAIR = 2 * VCHUNK
NPAIR = (VOCAB + PAIR - 1) // PAIR
VOCAB_PAD = NPAIR * PAIR
LASTBLK = (VOCAB - 1) // VCHUNK


def _tbody(a_ref, b_ref, out_ref):
    x = jnp.concatenate([a_ref[...], b_ref[...]], axis=0)
    out_ref[...] = x.T


def _relayout(table_t):
    return pl.pallas_call(
        _tbody,
        grid=(NPAIR,),
        in_specs=[
            pl.BlockSpec(
                (EMBED_DIM, VCHUNK), lambda j: (0, jnp.minimum(2 * j, LASTBLK))
            ),
            pl.BlockSpec(
                (EMBED_DIM, VCHUNK), lambda j: (0, jnp.minimum(2 * j + 1, LASTBLK))
            ),
        ],
        out_specs=pl.BlockSpec((VCHUNK, 2 * EMBED_DIM), lambda j: (j, 0)),
        out_shape=jax.ShapeDtypeStruct((NPAIR * VCHUNK, 2 * EMBED_DIM), jnp.float32),
    )(table_t, table_t)


def _slot64(v):
    j = v // PAIR
    r = v & (PAIR - 1)
    h = r // VCHUNK
    p = r & (VCHUNK - 1)
    return 2 * (j * VCHUNK + p) + h


def _body(idx_hbm, table_hbm, out_hbm, idx_v, rows_v, out_v, sems):
    c = lax.axis_index("c")
    s = lax.axis_index("s")
    wid = s * NC + c

    pltpu.sync_copy(idx_hbm.at[wid], idx_v)

    inv = jnp.float32(1.0 / HIST)

    def gather(g, b):
        return pltpu.make_async_copy(
            table_hbm.at[idx_v.at[g]], rows_v.at[b], sems.at[b]
        )

    for b in range(NBUF):
        gather(b, b).start()

    def outer(it, carry):
        for b in range(NBUF):
            g = it * NBUF + b
            gather(g, b).wait()
            for e in range(GRP):
                base = e * HIST
                accs = [
                    rows_v[b, base, pl.ds(k * NLANE, NLANE)] for k in range(KREG)
                ]
                for j in range(1, HIST):
                    for k in range(KREG):
                        accs[k] = accs[k] + rows_v[b, base + j, pl.ds(k * NLANE, NLANE)]
                for k in range(KREG):
                    out_v[g * GRP + e, pl.ds(k * NLANE, NLANE)] = accs[k] * inv

            @pl.when(g + NBUF < NGRP)
            def _():
                gather(g + NBUF, b).start()
        return carry

    lax.fori_loop(0, NGRP // NBUF, outer, 0)

    pltpu.sync_copy(out_v, out_hbm.at[wid])


def _gather_mean(idx3, table_half):
    mesh = plsc.VectorSubcoreMesh(core_axis_name="c", subcore_axis_name="s")
    f = pl.kernel(
        _body,
        out_type=jax.ShapeDtypeStruct((NW, EPW, HALF_DIM), jnp.float32),
        mesh=mesh,
        scratch_types=[
            pltpu.VMEM((NGRP, GRP * HIST), jnp.int32),
            pltpu.VMEM((NBUF, GRP * HIST, HALF_DIM), jnp.float32),
            pltpu.VMEM((EPW, HALF_DIM), jnp.float32),
            pltpu.SemaphoreType.DMA((NBUF,)),
        ],
        compiler_params=pltpu.CompilerParams(use_tc_tiling_on_sc=False),
    )
    return f(idx3, table_half)


@jax.jit
def _run(indices, table):
    s64 = _slot64(indices.astype(jnp.int32))
    idx_a = (2 * s64).reshape(NW, NGRP, GRP * HIST)
    idx_b = (2 * s64 + 1).reshape(NW, NGRP, GRP * HIST)
    packed = _relayout(table.T)
    t32 = packed.reshape(4 * NPAIR * VCHUNK, HALF_DIM)
    o0 = _gather_mean(idx_a, t32)
    o1 = _gather_mean(idx_b, t32)
    out = jnp.concatenate([o0, o1], axis=-1)
    return out.reshape(BATCH, EMBED_DIM)


def kernel(indices, table):
    return _run(indices, table)


# speedup vs baseline: 1.6670x; 1.0138x over previous
import jax
import jax.numpy as jnp
from jax import lax
from jax.experimental import pallas as pl
from jax.experimental.pallas import tpu as pltpu
from jax.experimental.pallas import tpu_sc as plsc

VOCAB = 1000000
EMBED_DIM = 64
HALF_DIM = EMBED_DIM // 2
BATCH = 16384
HIST = 50

NC = 2
NS = 16
NW = NC * NS
EPW = BATCH // NW
GRP = 2
NGRP = EPW // GRP
NBUF = 4
NLANE = 16
KREG = HALF_DIM // NLANE

VCHUNK = 16384
PAIR = 2 * VCHUNK
NPAIR = (VOCAB + PAIR - 1) // PAIR
VOCAB_PAD = NPAIR * PAIR
LASTBLK = (VOCAB - 1) // VCHUNK


def _tbody(a_ref, b_ref, out_ref):
    x = jnp.concatenate([a_ref[...], b_ref[...]], axis=0)
    out_ref[...] = x.T


def _relayout(table_t):
    return pl.pallas_call(
        _tbody,
        grid=(NPAIR,),
        in_specs=[
            pl.BlockSpec(
                (EMBED_DIM, VCHUNK), lambda j: (0, jnp.minimum(2 * j, LASTBLK))
            ),
            pl.BlockSpec(
                (EMBED_DIM, VCHUNK), lambda j: (0, jnp.minimum(2 * j + 1, LASTBLK))
            ),
        ],
        out_specs=pl.BlockSpec((VCHUNK, 2 * EMBED_DIM), lambda j: (j, 0)),
        out_shape=jax.ShapeDtypeStruct((NPAIR * VCHUNK, 2 * EMBED_DIM), jnp.float32),
    )(table_t, table_t)


def _slot64(v):
    j = v // PAIR
    r = v & (PAIR - 1)
    h = r // VCHUNK
    p = r & (VCHUNK - 1)
    return 2 * (j * VCHUNK + p) + h


def _body(idx_hbm, table_hbm, out_hbm, idx_v, rows_v, out_v, sems):
    c = lax.axis_index("c")
    s = lax.axis_index("s")
    wid = s * NC + c

    pltpu.sync_copy(idx_hbm.at[wid], idx_v)

    inv = jnp.float32(1.0 / HIST)

    def gather(g, b):
        return pltpu.make_async_copy(
            table_hbm.at[idx_v.at[g]], rows_v.at[b], sems.at[b]
        )

    for b in range(NBUF):
        gather(b, b).start()

    def outer(it, carry):
        for b in range(NBUF):
            g = it * NBUF + b
            gather(g, b).wait()
            for e in range(GRP):
                base = e * HIST
                accs = [
                    rows_v[b, base, pl.ds(k * NLANE, NLANE)] for k in range(KREG)
                ]
                for j in range(1, HIST):
                    for k in range(KREG):
                        accs[k] = accs[k] + rows_v[b, base + j, pl.ds(k * NLANE, NLANE)]
                for k in range(KREG):
                    out_v[g * GRP + e, pl.ds(k * NLANE, NLANE)] = accs[k] * inv

            @pl.when(g + NBUF < NGRP)
            def _():
                gather(g + NBUF, b).start()
        return carry

    lax.fori_loop(0, NGRP // NBUF, outer, 0)

    pltpu.sync_copy(out_v, out_hbm.at[wid])


def _gather_mean(idx3, table_half):
    mesh = plsc.VectorSubcoreMesh(core_axis_name="c", subcore_axis_name="s")
    f = pl.kernel(
        _body,
        out_type=jax.ShapeDtypeStruct((NW, EPW, HALF_DIM), jnp.float32),
        mesh=mesh,
        scratch_types=[
            pltpu.VMEM((NGRP, GRP * HIST), jnp.int32),
            pltpu.VMEM((NBUF, GRP * HIST, HALF_DIM), jnp.float32),
            pltpu.VMEM((EPW, HALF_DIM), jnp.float32),
            pltpu.SemaphoreType.DMA((NBUF,)),
        ],
        compiler_params=pltpu.CompilerParams(use_tc_tiling_on_sc=False),
    )
    return f(idx3, table_half)


@jax.jit
def _run(indices, table):
    s64 = _slot64(indices.astype(jnp.int32))
    idx_a = (2 * s64).reshape(NW, NGRP, GRP * HIST)
    idx_b = (2 * s64 + 1).reshape(NW, NGRP, GRP * HIST)
    packed = _relayout(table.T)
    t32 = packed.reshape(4 * NPAIR * VCHUNK, HALF_DIM)
    o0 = _gather_mean(idx_a, t32)
    o1 = _gather_mean(idx_b, t32)
    out = jnp.concatenate([o0, o1], axis=-1)
    return out.reshape(BATCH, EMBED_DIM)


def kernel(indices, table):
    return _run(indices, table)
